# Initial kernel scaffold; baseline (speedup 1.0000x reference)
import jax
import jax.numpy as jnp
from jax.experimental import pallas as pl


def kernel(x, shifts):
    raise NotImplementedError("write your pallas kernel here")


# speedup vs baseline: 4.2251x; 4.2251x over previous
import functools

import jax
import jax.numpy as jnp
from jax import lax
from jax.experimental import pallas as pl
from jax.experimental.pallas import tpu as pltpu
from jax.experimental.pallas import tpu_sc as plsc

_L = 1600
_B = 128
_T = 16000
_PAD_W = _T + 2 * _L
_NC = 2
_NS = 16
_NW = _NC * _NS
_ROWS_PER_W = _B // _NW
_LANES = 16
_CHUNKS = _T // _LANES


def _sc_shift_kernel(x_hbm, shifts_hbm, out_hbm, pad_v, out_v, sh_v):
  wid = lax.axis_index("s") * _NC + lax.axis_index("c")

  pltpu.sync_copy(shifts_hbm, sh_v.at[pl.ds(0, _B)])

  zeros = jnp.zeros((_LANES,), jnp.float32)
  for i in range(_L // _LANES):
    pad_v[pl.ds(i * _LANES, _LANES)] = zeros
    pad_v[pl.ds(_L + _T + i * _LANES, _LANES)] = zeros

  for j in range(_ROWS_PER_W):
    r = wid * _ROWS_PER_W + j
    pltpu.sync_copy(x_hbm.at[pl.ds(r * _T, _T)], pad_v.at[pl.ds(_L, _T)])
    start = (2 * _L) - sh_v[pl.ds(r, _LANES)][0]

    @plsc.parallel_loop(0, _CHUNKS, 1, unroll=8)
    def chunk(k):
      out_v[pl.ds(k * _LANES, _LANES)] = pad_v[pl.ds(start + k * _LANES,
                                                     _LANES)]

    pltpu.sync_copy(out_v, out_hbm.at[pl.ds(r * _T, _T)])


def kernel(x, shifts):
  mesh = plsc.VectorSubcoreMesh(core_axis_name="c", subcore_axis_name="s")
  f = functools.partial(
      pl.kernel,
      out_type=jax.ShapeDtypeStruct((_B * _T,), jnp.float32),
      mesh=mesh,
      scratch_types=[
          pltpu.VMEM((_PAD_W,), jnp.float32),
          pltpu.VMEM((_T,), jnp.float32),
          pltpu.VMEM((_B + _LANES,), jnp.int32),
      ],
  )(_sc_shift_kernel)
  return f(x.reshape(_B * _T), shifts).reshape(_B, _T)


# speedup vs baseline: 4.5044x; 1.0661x over previous
import functools

import jax
import jax.numpy as jnp
from jax import lax
from jax.experimental import pallas as pl
from jax.experimental.pallas import tpu as pltpu
from jax.experimental.pallas import tpu_sc as plsc

_L = 1600
_B = 128
_T = 16000
_PAD_W = _T + 2 * _L
_NC = 2
_NS = 16
_NW = _NC * _NS
_ROWS_PER_W = _B // _NW
_LANES = 16
_CHUNKS = _T // _LANES


def _sc_shift_kernel(x_hbm, shifts_hbm, out_hbm, pad0, pad1, out0, out1,
                     sh_v, in_sems, out_sems):
  wid = lax.axis_index("s") * _NC + lax.axis_index("c")
  pads = [pad0, pad1]
  outs = [out0, out1]

  pltpu.sync_copy(shifts_hbm, sh_v.at[pl.ds(0, _B)])

  zeros = jnp.zeros((_LANES,), jnp.float32)
  for b in range(2):
    for i in range(_L // _LANES):
      pads[b][pl.ds(i * _LANES, _LANES)] = zeros
      pads[b][pl.ds(_L + _T + i * _LANES, _LANES)] = zeros

  def fire_in(j):
    r = wid * _ROWS_PER_W + j
    return pltpu.async_copy(x_hbm.at[pl.ds(r * _T, _T)],
                            pads[j % 2].at[pl.ds(_L, _T)], in_sems[j % 2])

  in_handles = [fire_in(0), fire_in(1)]
  out_handles = [None, None]

  for j in range(_ROWS_PER_W):
    b = j % 2
    r = wid * _ROWS_PER_W + j
    if out_handles[b] is not None:
      out_handles[b].wait()
    in_handles[b].wait()
    start = (2 * _L) - sh_v[pl.ds(r, _LANES)][0]
    pad_b, out_b = pads[b], outs[b]

    @plsc.parallel_loop(0, _CHUNKS, 1, unroll=8)
    def chunk(k):
      out_b[pl.ds(k * _LANES, _LANES)] = pad_b[pl.ds(start + k * _LANES,
                                                     _LANES)]

    out_handles[b] = pltpu.async_copy(out_b, out_hbm.at[pl.ds(r * _T, _T)],
                                      out_sems[b])
    if j + 2 < _ROWS_PER_W:
      in_handles[b] = fire_in(j + 2)

  out_handles[0].wait()
  out_handles[1].wait()


def kernel(x, shifts):
  mesh = plsc.VectorSubcoreMesh(core_axis_name="c", subcore_axis_name="s")
  f = functools.partial(
      pl.kernel,
      out_type=jax.ShapeDtypeStruct((_B * _T,), jnp.float32),
      mesh=mesh,
      scratch_types=[
          pltpu.VMEM((_PAD_W,), jnp.float32),
          pltpu.VMEM((_PAD_W,), jnp.float32),
          pltpu.VMEM((_T,), jnp.float32),
          pltpu.VMEM((_T,), jnp.float32),
          pltpu.VMEM((_B + _LANES,), jnp.int32),
          [pltpu.SemaphoreType.DMA] * 2,
          [pltpu.SemaphoreType.DMA] * 2,
      ],
  )(_sc_shift_kernel)
  return f(x.reshape(_B * _T), shifts).reshape(_B, _T)
